# Initial kernel scaffold; baseline (speedup 1.0000x reference)
import jax
import jax.numpy as jnp
from jax.experimental import pallas as pl


def kernel(x, edge_index, edge_label_index, W1, b1, W2, b2):
    raise NotImplementedError("write your pallas kernel here")


# speedup vs baseline: 11.7747x; 11.7747x over previous
import functools

import jax
import jax.numpy as jnp
from jax import lax
from jax.experimental import pallas as pl
from jax.experimental.pallas import tpu as pltpu
from jax.experimental.pallas import tpu_sc as plsc

N = 10000
D = 128
H = 128
E = 320000
EL = 100000

N_PAD = 10240
PAD_ROWS = N_PAD - N

NC, NS = 2, 16
NW = NC * NS
CHUNK = 128

E_TILE = 10112
E_PAD = NW * E_TILE
L_TILE = 3200
EL_PAD = NW * L_TILE

ROWS_PER_TILE = N_PAD // NS


def _worker_id():
    return lax.axis_index("s") * NC + lax.axis_index("c")


def _sc_deg_body(dst_hbm, zrow_hbm, orow_hbm, out_hbm, idx_v, ones_v, deg_sh, sem):
    cid = lax.axis_index("c")
    sid = lax.axis_index("s")
    wid = _worker_id()
    for j in range(ROWS_PER_TILE // CHUNK):
        pltpu.sync_copy(zrow_hbm, deg_sh.at[pl.ds(sid * ROWS_PER_TILE + j * CHUNK, CHUNK)])
    pltpu.sync_copy(orow_hbm, ones_v)
    plsc.subcore_barrier()

    def chunk(c, carry):
        off = wid * E_TILE + c * CHUNK
        pltpu.sync_copy(dst_hbm.at[pl.ds(off, CHUNK)], idx_v)
        pltpu.sync_copy(ones_v, deg_sh.at[idx_v], add=True)
        return carry

    lax.fori_loop(0, E_TILE // CHUNK, chunk, 0)
    plsc.subcore_barrier()
    pltpu.sync_copy(
        deg_sh.at[pl.ds(sid * ROWS_PER_TILE, ROWS_PER_TILE)],
        out_hbm.at[pl.ds(cid * N_PAD + sid * ROWS_PER_TILE, ROWS_PER_TILE)],
    )


_sc_deg = pl.kernel(
    _sc_deg_body,
    out_type=jax.ShapeDtypeStruct((NC * N_PAD, H), jnp.float32),
    mesh=plsc.VectorSubcoreMesh(core_axis_name="c", subcore_axis_name="s"),
    scratch_types=[
        pltpu.VMEM((CHUNK,), jnp.int32),
        pltpu.VMEM((CHUNK, H), jnp.float32),
        pltpu.VMEM_SHARED((N_PAD, H), jnp.float32),
        pltpu.SemaphoreType.DMA,
    ],
)


def _sc_msg_body(g_hbm, src_hbm, dst_hbm, zrow_hbm, out_hbm,
                 sidx_v, didx_v, rows_v, acc_sh, sem):
    cid = lax.axis_index("c")
    sid = lax.axis_index("s")
    wid = _worker_id()
    for j in range(ROWS_PER_TILE // CHUNK):
        pltpu.sync_copy(zrow_hbm, acc_sh.at[pl.ds(sid * ROWS_PER_TILE + j * CHUNK, CHUNK)])
    plsc.subcore_barrier()

    def chunk(c, carry):
        off = wid * E_TILE + c * CHUNK
        pltpu.sync_copy(src_hbm.at[pl.ds(off, CHUNK)], sidx_v)
        pltpu.sync_copy(dst_hbm.at[pl.ds(off, CHUNK)], didx_v)
        pltpu.async_copy(g_hbm.at[sidx_v], rows_v, sem).wait()
        pltpu.sync_copy(rows_v, acc_sh.at[didx_v], add=True)
        return carry

    lax.fori_loop(0, E_TILE // CHUNK, chunk, 0)
    plsc.subcore_barrier()
    pltpu.sync_copy(
        acc_sh.at[pl.ds(sid * ROWS_PER_TILE, ROWS_PER_TILE)],
        out_hbm.at[pl.ds(cid * N_PAD + sid * ROWS_PER_TILE, ROWS_PER_TILE)],
    )


_sc_msg = pl.kernel(
    _sc_msg_body,
    out_type=jax.ShapeDtypeStruct((NC * N_PAD, H), jnp.float32),
    mesh=plsc.VectorSubcoreMesh(core_axis_name="c", subcore_axis_name="s"),
    scratch_types=[
        pltpu.VMEM((CHUNK,), jnp.int32),
        pltpu.VMEM((CHUNK,), jnp.int32),
        pltpu.VMEM((CHUNK, H), jnp.float32),
        pltpu.VMEM_SHARED((N_PAD, H), jnp.float32),
        pltpu.SemaphoreType.DMA,
    ],
)


def _sc_decode_body(z_hbm, ea_hbm, eb_hbm, out_hbm,
                    aidx_v, bidx_v, rowsa_v, rowsb_v, ps_v, sema, semb):
    wid = _worker_id()

    def chunk(c, carry):
        off = wid * L_TILE + c * CHUNK
        pltpu.sync_copy(ea_hbm.at[pl.ds(off, CHUNK)], aidx_v)
        pltpu.sync_copy(eb_hbm.at[pl.ds(off, CHUNK)], bidx_v)
        cpa = pltpu.async_copy(z_hbm.at[aidx_v], rowsa_v, sema)
        cpb = pltpu.async_copy(z_hbm.at[bidx_v], rowsb_v, semb)
        cpa.wait()
        cpb.wait()

        def edge(e, carry2):
            acc = rowsa_v[e, pl.ds(0, 16)] * rowsb_v[e, pl.ds(0, 16)]
            for j in range(1, H // 16):
                acc = acc + rowsa_v[e, pl.ds(j * 16, 16)] * rowsb_v[e, pl.ds(j * 16, 16)]
            ps_v[e, :] = acc
            return carry2

        lax.fori_loop(0, CHUNK, edge, 0)
        pltpu.sync_copy(ps_v, out_hbm.at[pl.ds(off, CHUNK)])
        return carry

    lax.fori_loop(0, L_TILE // CHUNK, chunk, 0)


_sc_decode = pl.kernel(
    _sc_decode_body,
    out_type=jax.ShapeDtypeStruct((EL_PAD, 16), jnp.float32),
    mesh=plsc.VectorSubcoreMesh(core_axis_name="c", subcore_axis_name="s"),
    scratch_types=[
        pltpu.VMEM((CHUNK,), jnp.int32),
        pltpu.VMEM((CHUNK,), jnp.int32),
        pltpu.VMEM((CHUNK, H), jnp.float32),
        pltpu.VMEM((CHUNK, H), jnp.float32),
        pltpu.VMEM((CHUNK, 16), jnp.float32),
        pltpu.SemaphoreType.DMA,
        pltpu.SemaphoreType.DMA,
    ],
)


ROW_BLK = 512
GRID_N = N_PAD // ROW_BLK


def _dis_block(deg_blk):
    deg = 1.0 + deg_blk[0, :, 0:1] + deg_blk[1, :, 0:1]
    return lax.rsqrt(deg)


def _tc_prep1_body(x_ref, w_ref, deg_ref, g_ref):
    dis = _dis_block(deg_ref[...])
    h = jnp.dot(x_ref[...], w_ref[...], preferred_element_type=jnp.float32)
    g_ref[...] = h * dis


_tc_prep1 = pl.pallas_call(
    _tc_prep1_body,
    out_shape=jax.ShapeDtypeStruct((N_PAD, H), jnp.float32),
    grid=(GRID_N,),
    in_specs=[
        pl.BlockSpec((ROW_BLK, D), lambda i: (i, 0)),
        pl.BlockSpec((D, H), lambda i: (0, 0)),
        pl.BlockSpec((2, ROW_BLK, 8), lambda i: (0, i, 0)),
    ],
    out_specs=pl.BlockSpec((ROW_BLK, H), lambda i: (i, 0)),
)


def _tc_prep2_body(p_ref, g_ref, deg_ref, b_ref, w_ref, o_ref):
    dis = _dis_block(deg_ref[...])
    p = p_ref[...]
    z = jnp.maximum(dis * (p[0] + p[1] + g_ref[...]) + b_ref[...], 0.0)
    o_ref[...] = jnp.dot(z, w_ref[...], preferred_element_type=jnp.float32) * dis


_tc_prep2 = pl.pallas_call(
    _tc_prep2_body,
    out_shape=jax.ShapeDtypeStruct((N_PAD, H), jnp.float32),
    grid=(GRID_N,),
    in_specs=[
        pl.BlockSpec((2, ROW_BLK, H), lambda i: (0, i, 0)),
        pl.BlockSpec((ROW_BLK, H), lambda i: (i, 0)),
        pl.BlockSpec((2, ROW_BLK, 8), lambda i: (0, i, 0)),
        pl.BlockSpec((1, H), lambda i: (0, 0)),
        pl.BlockSpec((H, H), lambda i: (0, 0)),
    ],
    out_specs=pl.BlockSpec((ROW_BLK, H), lambda i: (i, 0)),
)


def _tc_final_body(p_ref, g_ref, deg_ref, b_ref, o_ref):
    dis = _dis_block(deg_ref[...])
    p = p_ref[...]
    o_ref[...] = dis * (p[0] + p[1] + g_ref[...]) + b_ref[...]


_tc_final = pl.pallas_call(
    _tc_final_body,
    out_shape=jax.ShapeDtypeStruct((N_PAD, H), jnp.float32),
    grid=(GRID_N,),
    in_specs=[
        pl.BlockSpec((2, ROW_BLK, H), lambda i: (0, i, 0)),
        pl.BlockSpec((ROW_BLK, H), lambda i: (i, 0)),
        pl.BlockSpec((2, ROW_BLK, 8), lambda i: (0, i, 0)),
        pl.BlockSpec((1, H), lambda i: (0, 0)),
    ],
    out_specs=pl.BlockSpec((ROW_BLK, H), lambda i: (i, 0)),
)


RED_BLK = 16
RED_ROWS = EL_PAD // 128


def _tc_reduce_body(ps_ref, o_ref):
    o_ref[...] = jnp.sum(ps_ref[...], axis=-1)


_tc_reduce = pl.pallas_call(
    _tc_reduce_body,
    out_shape=jax.ShapeDtypeStruct((RED_ROWS, 128), jnp.float32),
    grid=(RED_ROWS // RED_BLK,),
    in_specs=[pl.BlockSpec((RED_BLK, 128, 16), lambda i: (i, 0, 0))],
    out_specs=pl.BlockSpec((RED_BLK, 128), lambda i: (i, 0)),
)


def kernel(x, edge_index, edge_label_index, W1, b1, W2, b2):
    src, dst = edge_index[0], edge_index[1]
    epad = N + (jnp.arange(E_PAD - E, dtype=jnp.int32) % PAD_ROWS)
    srcp = jnp.concatenate([src, epad])
    dstp = jnp.concatenate([dst, epad])
    lpad = N + (jnp.arange(EL_PAD - EL, dtype=jnp.int32) % PAD_ROWS)
    ea = jnp.concatenate([edge_label_index[0], lpad])
    eb = jnp.concatenate([edge_label_index[1], lpad])
    x_pad = jnp.zeros((N_PAD, D), jnp.float32).at[:N].set(x)
    zrow = jnp.zeros((CHUNK, H), jnp.float32)
    orow = jnp.ones((CHUNK, H), jnp.float32)

    degparts = _sc_deg(dstp, zrow, orow).reshape(NC, N_PAD, H)[:, :, :8]
    g1 = _tc_prep1(x_pad, W1, degparts)
    p1 = _sc_msg(g1, srcp, dstp, zrow).reshape(NC, N_PAD, H)
    g2 = _tc_prep2(p1, g1, degparts, b1.reshape(1, H), W2)
    p2 = _sc_msg(g2, srcp, dstp, zrow).reshape(NC, N_PAD, H)
    z2 = _tc_final(p2, g2, degparts, b2.reshape(1, H))
    ps = _sc_decode(z2, ea, eb)
    red = _tc_reduce(ps.reshape(RED_ROWS, 128, 16))
    return red.reshape(EL_PAD)[:EL]


# speedup vs baseline: 16.5892x; 1.4089x over previous
import functools

import jax
import jax.numpy as jnp
from jax import lax
from jax.experimental import pallas as pl
from jax.experimental.pallas import tpu as pltpu
from jax.experimental.pallas import tpu_sc as plsc

N = 10000
D = 128
H = 128
E = 320000
EL = 100000

N_PAD = 10240
PAD_ROWS = N_PAD - N

NC, NS = 2, 16
NW = NC * NS
CHUNK = 128

E_TILE = 10240
E_PAD = NW * E_TILE
E_CH = E_TILE // CHUNK
L_TILE = 3200
EL_PAD = NW * L_TILE

ROWS_PER_TILE = N_PAD // NS


def _worker_id():
    return lax.axis_index("s") * NC + lax.axis_index("c")


def _fill_const(buf, val):
    v = jnp.full((16,), val, jnp.float32)

    def row(i, carry):
        for j in range(H // 16):
            buf[i, pl.ds(j * 16, 16)] = v
        return carry

    lax.fori_loop(0, CHUNK, row, 0)


def _zero_acc_slice(buf, acc_sh, sid):
    for j in range(ROWS_PER_TILE // CHUNK):
        pltpu.sync_copy(buf, acc_sh.at[pl.ds(sid * ROWS_PER_TILE + j * CHUNK, CHUNK)])


def _row_iota(buf, base, n):
    for j in range(n // 16):
        buf[pl.ds(j * 16, 16)] = base + j * 16 + lax.iota(jnp.int32, 16)


def _sc_deg_body(dst_hbm, out_hbm, ridx_v, didx_v, ones_v, deg_sh, sem):
    cid = lax.axis_index("c")
    sid = lax.axis_index("s")
    wid = _worker_id()
    _fill_const(ones_v, 0.0)
    _zero_acc_slice(ones_v, deg_sh, sid)
    _fill_const(ones_v, 1.0)
    _row_iota(ridx_v, wid * E_CH, E_CH)
    pltpu.async_copy(dst_hbm.at[ridx_v], didx_v, sem).wait()
    plsc.subcore_barrier()

    GRP = 4

    def group(g, carry):
        for k in range(GRP):
            pltpu.async_copy(ones_v, deg_sh.at[didx_v.at[g * GRP + k]], sem, add=True)
        for k in range(GRP):
            pltpu.make_async_copy(ones_v, deg_sh.at[didx_v.at[0]], sem).wait()
        return carry

    lax.fori_loop(0, E_CH // GRP, group, 0)
    plsc.subcore_barrier()
    pltpu.sync_copy(
        deg_sh.at[pl.ds(sid * ROWS_PER_TILE, ROWS_PER_TILE)],
        out_hbm.at[pl.ds(cid * N_PAD + sid * ROWS_PER_TILE, ROWS_PER_TILE)],
    )


_sc_deg = pl.kernel(
    _sc_deg_body,
    out_type=jax.ShapeDtypeStruct((NC * N_PAD, H), jnp.float32),
    mesh=plsc.VectorSubcoreMesh(core_axis_name="c", subcore_axis_name="s"),
    scratch_types=[
        pltpu.VMEM((E_CH,), jnp.int32),
        pltpu.VMEM((E_CH, CHUNK), jnp.int32),
        pltpu.VMEM((CHUNK, H), jnp.float32),
        pltpu.VMEM_SHARED((N_PAD, H), jnp.float32),
        pltpu.SemaphoreType.DMA,
    ],
)


def _sc_msg_body(g_hbm, src_hbm, dst_hbm, out_hbm,
                 sidx0_v, sidx1_v, didx0_v, didx1_v, rows0_v, rows1_v, acc_sh,
                 semi0, semi1, semg0, semg1):
    cid = lax.axis_index("c")
    sid = lax.axis_index("s")
    wid = _worker_id()
    _fill_const(rows0_v, 0.0)
    _zero_acc_slice(rows0_v, acc_sh, sid)
    base = wid * E_TILE

    def idxload(c, sbuf, dbuf, sem):
        pltpu.async_copy(src_hbm.at[pl.ds(base + c * CHUNK, CHUNK)], sbuf, sem)
        pltpu.async_copy(dst_hbm.at[pl.ds(base + c * CHUNK, CHUNK)], dbuf, sem)

    def idxwait(sbuf, dbuf, sem):
        pltpu.make_async_copy(src_hbm.at[pl.ds(0, CHUNK)], sbuf, sem).wait()
        pltpu.make_async_copy(dst_hbm.at[pl.ds(0, CHUNK)], dbuf, sem).wait()

    def gather(sbuf, rbuf, sem):
        pltpu.async_copy(g_hbm.at[sbuf], rbuf, sem)

    def gwait(rbuf, sem):
        pltpu.make_async_copy(g_hbm.at[sidx0_v], rbuf, sem).wait()

    def scat(dbuf, rbuf):
        pltpu.sync_copy(rbuf, acc_sh.at[dbuf], add=True)

    idxload(0, sidx0_v, didx0_v, semi0)
    idxwait(sidx0_v, didx0_v, semi0)
    gather(sidx0_v, rows0_v, semg0)
    idxload(1, sidx1_v, didx1_v, semi1)

    def pair(i, carry):
        c0 = 2 * i
        last = i >= E_CH // 2 - 1
        idxwait(sidx1_v, didx1_v, semi1)
        gwait(rows0_v, semg0)
        gather(sidx1_v, rows1_v, semg1)
        scat(didx0_v, rows0_v)

        @pl.when(jnp.logical_not(last))
        def _():
            idxload(c0 + 2, sidx0_v, didx0_v, semi0)

        gwait(rows1_v, semg1)
        scat(didx1_v, rows1_v)

        @pl.when(jnp.logical_not(last))
        def _():
            idxwait(sidx0_v, didx0_v, semi0)
            gather(sidx0_v, rows0_v, semg0)
            idxload(c0 + 3, sidx1_v, didx1_v, semi1)

        return carry

    lax.fori_loop(0, E_CH // 2, pair, 0)
    plsc.subcore_barrier()
    pltpu.sync_copy(
        acc_sh.at[pl.ds(sid * ROWS_PER_TILE, ROWS_PER_TILE)],
        out_hbm.at[pl.ds(cid * N_PAD + sid * ROWS_PER_TILE, ROWS_PER_TILE)],
    )


_sc_msg = pl.kernel(
    _sc_msg_body,
    out_type=jax.ShapeDtypeStruct((NC * N_PAD, H), jnp.float32),
    mesh=plsc.VectorSubcoreMesh(core_axis_name="c", subcore_axis_name="s"),
    scratch_types=[
        pltpu.VMEM((CHUNK,), jnp.int32),
        pltpu.VMEM((CHUNK,), jnp.int32),
        pltpu.VMEM((CHUNK,), jnp.int32),
        pltpu.VMEM((CHUNK,), jnp.int32),
        pltpu.VMEM((CHUNK, H), jnp.float32),
        pltpu.VMEM((CHUNK, H), jnp.float32),
        pltpu.VMEM_SHARED((N_PAD, H), jnp.float32),
        pltpu.SemaphoreType.DMA,
        pltpu.SemaphoreType.DMA,
        pltpu.SemaphoreType.DMA,
        pltpu.SemaphoreType.DMA,
    ],
)


def _sc_decode_body(z_hbm, ea_hbm, eb_hbm, out_hbm,
                    aidx_v, bidx_v, rowsa_v, rowsb_v, ps_v, sema, semb):
    wid = _worker_id()

    def chunk(c, carry):
        off = wid * L_TILE + c * CHUNK
        pltpu.sync_copy(ea_hbm.at[pl.ds(off, CHUNK)], aidx_v)
        pltpu.sync_copy(eb_hbm.at[pl.ds(off, CHUNK)], bidx_v)
        cpa = pltpu.async_copy(z_hbm.at[aidx_v], rowsa_v, sema)
        cpb = pltpu.async_copy(z_hbm.at[bidx_v], rowsb_v, semb)
        cpa.wait()
        cpb.wait()

        def edge(e, carry2):
            acc = rowsa_v[e, pl.ds(0, 16)] * rowsb_v[e, pl.ds(0, 16)]
            for j in range(1, H // 16):
                acc = acc + rowsa_v[e, pl.ds(j * 16, 16)] * rowsb_v[e, pl.ds(j * 16, 16)]
            ps_v[e, :] = acc
            return carry2

        lax.fori_loop(0, CHUNK, edge, 0)
        pltpu.sync_copy(ps_v, out_hbm.at[pl.ds(off, CHUNK)])
        return carry

    lax.fori_loop(0, L_TILE // CHUNK, chunk, 0)


_sc_decode = pl.kernel(
    _sc_decode_body,
    out_type=jax.ShapeDtypeStruct((EL_PAD, 16), jnp.float32),
    mesh=plsc.VectorSubcoreMesh(core_axis_name="c", subcore_axis_name="s"),
    scratch_types=[
        pltpu.VMEM((CHUNK,), jnp.int32),
        pltpu.VMEM((CHUNK,), jnp.int32),
        pltpu.VMEM((CHUNK, H), jnp.float32),
        pltpu.VMEM((CHUNK, H), jnp.float32),
        pltpu.VMEM((CHUNK, 16), jnp.float32),
        pltpu.SemaphoreType.DMA,
        pltpu.SemaphoreType.DMA,
    ],
)


ROW_BLK = 512
GRID_N = N_PAD // ROW_BLK


def _dis_block(deg_blk):
    deg = 1.0 + deg_blk[0, :, 0:1] + deg_blk[1, :, 0:1]
    return lax.rsqrt(deg)


def _tc_prep1_body(x_ref, w_ref, deg_ref, g_ref):
    dis = _dis_block(deg_ref[...])
    h = jnp.dot(x_ref[...], w_ref[...], preferred_element_type=jnp.float32)
    g_ref[...] = h * dis


_tc_prep1 = pl.pallas_call(
    _tc_prep1_body,
    out_shape=jax.ShapeDtypeStruct((N_PAD, H), jnp.float32),
    grid=(GRID_N,),
    in_specs=[
        pl.BlockSpec((ROW_BLK, D), lambda i: (i, 0)),
        pl.BlockSpec((D, H), lambda i: (0, 0)),
        pl.BlockSpec((2, ROW_BLK, 8), lambda i: (0, i, 0)),
    ],
    out_specs=pl.BlockSpec((ROW_BLK, H), lambda i: (i, 0)),
)


def _tc_prep2_body(p_ref, g_ref, deg_ref, b_ref, w_ref, o_ref):
    dis = _dis_block(deg_ref[...])
    p = p_ref[...]
    z = jnp.maximum(dis * (p[0] + p[1] + g_ref[...]) + b_ref[...], 0.0)
    o_ref[...] = jnp.dot(z, w_ref[...], preferred_element_type=jnp.float32) * dis


_tc_prep2 = pl.pallas_call(
    _tc_prep2_body,
    out_shape=jax.ShapeDtypeStruct((N_PAD, H), jnp.float32),
    grid=(GRID_N,),
    in_specs=[
        pl.BlockSpec((2, ROW_BLK, H), lambda i: (0, i, 0)),
        pl.BlockSpec((ROW_BLK, H), lambda i: (i, 0)),
        pl.BlockSpec((2, ROW_BLK, 8), lambda i: (0, i, 0)),
        pl.BlockSpec((1, H), lambda i: (0, 0)),
        pl.BlockSpec((H, H), lambda i: (0, 0)),
    ],
    out_specs=pl.BlockSpec((ROW_BLK, H), lambda i: (i, 0)),
)


def _tc_final_body(p_ref, g_ref, deg_ref, b_ref, o_ref):
    dis = _dis_block(deg_ref[...])
    p = p_ref[...]
    o_ref[...] = dis * (p[0] + p[1] + g_ref[...]) + b_ref[...]


_tc_final = pl.pallas_call(
    _tc_final_body,
    out_shape=jax.ShapeDtypeStruct((N_PAD, H), jnp.float32),
    grid=(GRID_N,),
    in_specs=[
        pl.BlockSpec((2, ROW_BLK, H), lambda i: (0, i, 0)),
        pl.BlockSpec((ROW_BLK, H), lambda i: (i, 0)),
        pl.BlockSpec((2, ROW_BLK, 8), lambda i: (0, i, 0)),
        pl.BlockSpec((1, H), lambda i: (0, 0)),
    ],
    out_specs=pl.BlockSpec((ROW_BLK, H), lambda i: (i, 0)),
)


RED_BLK = 16
RED_ROWS = EL_PAD // 128


def _tc_reduce_body(ps_ref, o_ref):
    o_ref[...] = jnp.sum(ps_ref[...], axis=-1)


_tc_reduce = pl.pallas_call(
    _tc_reduce_body,
    out_shape=jax.ShapeDtypeStruct((RED_ROWS, 128), jnp.float32),
    grid=(RED_ROWS // RED_BLK,),
    in_specs=[pl.BlockSpec((RED_BLK, 128, 16), lambda i: (i, 0, 0))],
    out_specs=pl.BlockSpec((RED_BLK, 128), lambda i: (i, 0)),
)


def kernel(x, edge_index, edge_label_index, W1, b1, W2, b2):
    src, dst = edge_index[0], edge_index[1]
    epad = N + (jnp.arange(E_PAD - E, dtype=jnp.int32) % PAD_ROWS)
    srcp = jnp.concatenate([src, epad])
    dstp = jnp.concatenate([dst, epad])
    lpad = N + (jnp.arange(EL_PAD - EL, dtype=jnp.int32) % PAD_ROWS)
    ea = jnp.concatenate([edge_label_index[0], lpad])
    eb = jnp.concatenate([edge_label_index[1], lpad])
    x_pad = jnp.zeros((N_PAD, D), jnp.float32).at[:N].set(x)

    degparts = _sc_deg(dstp.reshape(NW * E_CH, CHUNK)).reshape(NC, N_PAD, H)[:, :, :8]
    g1 = _tc_prep1(x_pad, W1, degparts)
    p1 = _sc_msg(g1, srcp, dstp).reshape(NC, N_PAD, H)
    g2 = _tc_prep2(p1, g1, degparts, b1.reshape(1, H), W2)
    p2 = _sc_msg(g2, srcp, dstp).reshape(NC, N_PAD, H)
    z2 = _tc_final(p2, g2, degparts, b2.reshape(1, H))
    ps = _sc_decode(z2, ea, eb)
    red = _tc_reduce(ps.reshape(RED_ROWS, 128, 16))
    return red.reshape(EL_PAD)[:EL]


# speedup vs baseline: 19.0140x; 1.1462x over previous
import functools

import jax
import jax.numpy as jnp
from jax import lax
from jax.experimental import pallas as pl
from jax.experimental.pallas import tpu as pltpu
from jax.experimental.pallas import tpu_sc as plsc

N = 10000
D = 128
H = 128
E = 320000
EL = 100000

N_PAD = 10240
PAD_ROWS = N_PAD - N

NC, NS = 2, 16
NW = NC * NS
CHUNK = 128

E_TILE = 10240
E_PAD = NW * E_TILE
E_CH = E_TILE // CHUNK
L_TILE = 3200
EL_PAD = NW * L_TILE

ROWS_PER_TILE = N_PAD // NS

MCHUNK = 128
M_CH = 80
M_TILE = M_CH * MCHUNK
M_PAD = NW * M_TILE


def _worker_id():
    return lax.axis_index("s") * NC + lax.axis_index("c")


def _fill_const(buf, val):
    v = jnp.full((16,), val, jnp.float32)
    n = buf.shape[0]

    def row(i, carry):
        for j in range(H // 16):
            buf[i, pl.ds(j * 16, 16)] = v
        return carry

    lax.fori_loop(0, n, row, 0)


def _zero_acc_slice(buf, acc_sh, sid):
    for j in range(ROWS_PER_TILE // CHUNK):
        pltpu.sync_copy(buf, acc_sh.at[pl.ds(sid * ROWS_PER_TILE + j * CHUNK, CHUNK)])


def _row_iota(buf, base, n):
    for j in range(n // 16):
        buf[pl.ds(j * 16, 16)] = base + j * 16 + lax.iota(jnp.int32, 16)


def _sc_deg_body(dst_hbm, out_hbm, ridx_v, didx_v, ones_v, deg_sh, sem):
    cid = lax.axis_index("c")
    sid = lax.axis_index("s")
    wid = _worker_id()
    _fill_const(ones_v, 0.0)
    _zero_acc_slice(ones_v, deg_sh, sid)
    _fill_const(ones_v, 1.0)
    _row_iota(ridx_v, wid * E_CH, E_CH)
    pltpu.async_copy(dst_hbm.at[ridx_v], didx_v, sem).wait()
    plsc.subcore_barrier()

    GRP = 4

    def group(g, carry):
        for k in range(GRP):
            pltpu.async_copy(ones_v, deg_sh.at[didx_v.at[g * GRP + k]], sem, add=True)
        for k in range(GRP):
            pltpu.make_async_copy(ones_v, deg_sh.at[didx_v.at[0]], sem).wait()
        return carry

    lax.fori_loop(0, E_CH // GRP, group, 0)
    plsc.subcore_barrier()
    pltpu.sync_copy(
        deg_sh.at[pl.ds(sid * ROWS_PER_TILE, ROWS_PER_TILE)],
        out_hbm.at[pl.ds(cid * N_PAD + sid * ROWS_PER_TILE, ROWS_PER_TILE)],
    )


_sc_deg = pl.kernel(
    _sc_deg_body,
    out_type=jax.ShapeDtypeStruct((NC * N_PAD, H), jnp.float32),
    mesh=plsc.VectorSubcoreMesh(core_axis_name="c", subcore_axis_name="s"),
    scratch_types=[
        pltpu.VMEM((E_CH,), jnp.int32),
        pltpu.VMEM((E_CH, CHUNK), jnp.int32),
        pltpu.VMEM((CHUNK, H), jnp.float32),
        pltpu.VMEM_SHARED((N_PAD, H), jnp.float32),
        pltpu.SemaphoreType.DMA,
    ],
)


def _sc_msg_body(g_hbm, pk_hbm, out_hbm,
                 pk_v, sidx0_v, sidx1_v, didx0_v, didx1_v, rows0_v, rows1_v,
                 acc_sh, semg0, semg1, sems0, sems1):
    cid = lax.axis_index("c")
    sid = lax.axis_index("s")
    wid = _worker_id()
    _fill_const(rows0_v, 0.0)
    _zero_acc_slice(rows0_v, acc_sh, sid)
    pltpu.sync_copy(pk_hbm.at[wid], pk_v)
    plsc.subcore_barrier()

    def unpack(c, sbuf, dbuf):
        for j in range(MCHUNK // 16):
            p = pk_v[c, pl.ds(j * 16, 16)]
            sbuf[pl.ds(j * 16, 16)] = p & 0xFFFF
            dbuf[pl.ds(j * 16, 16)] = lax.shift_right_logical(p, 16)

    def gather(sbuf, rbuf, sem):
        pltpu.async_copy(g_hbm.at[sbuf], rbuf, sem)

    def gwait(rbuf, sem):
        pltpu.make_async_copy(g_hbm.at[sidx0_v], rbuf, sem).wait()

    def sissue(dbuf, rbuf, sem):
        pltpu.async_copy(rbuf, acc_sh.at[dbuf], sem, add=True)

    def swait(sem):
        pltpu.make_async_copy(rows0_v, acc_sh.at[didx0_v], sem).wait()

    unpack(0, sidx0_v, didx0_v)
    gather(sidx0_v, rows0_v, semg0)

    def pair(i, carry):
        c0 = 2 * i

        @pl.when(i > 0)
        def _():
            swait(sems1)

        unpack(c0 + 1, sidx1_v, didx1_v)
        gather(sidx1_v, rows1_v, semg1)
        gwait(rows0_v, semg0)
        sissue(didx0_v, rows0_v, sems0)
        gwait(rows1_v, semg1)
        swait(sems0)

        @pl.when(i < M_CH // 2 - 1)
        def _():
            unpack(c0 + 2, sidx0_v, didx0_v)
            gather(sidx0_v, rows0_v, semg0)

        sissue(didx1_v, rows1_v, sems1)
        return carry

    lax.fori_loop(0, M_CH // 2, pair, 0)
    swait(sems1)
    plsc.subcore_barrier()
    pltpu.sync_copy(
        acc_sh.at[pl.ds(sid * ROWS_PER_TILE, ROWS_PER_TILE)],
        out_hbm.at[pl.ds(cid * N_PAD + sid * ROWS_PER_TILE, ROWS_PER_TILE)],
    )


_sc_msg = pl.kernel(
    _sc_msg_body,
    out_type=jax.ShapeDtypeStruct((NC * N_PAD, H), jnp.float32),
    mesh=plsc.VectorSubcoreMesh(core_axis_name="c", subcore_axis_name="s"),
    scratch_types=[
        pltpu.VMEM((M_CH, MCHUNK), jnp.int32),
        pltpu.VMEM((MCHUNK,), jnp.int32),
        pltpu.VMEM((MCHUNK,), jnp.int32),
        pltpu.VMEM((MCHUNK,), jnp.int32),
        pltpu.VMEM((MCHUNK,), jnp.int32),
        pltpu.VMEM((MCHUNK, H), jnp.float32),
        pltpu.VMEM((MCHUNK, H), jnp.float32),
        pltpu.VMEM_SHARED((N_PAD, H), jnp.float32),
        pltpu.SemaphoreType.DMA,
        pltpu.SemaphoreType.DMA,
        pltpu.SemaphoreType.DMA,
        pltpu.SemaphoreType.DMA,
    ],
)


def _sc_decode_body(z_hbm, ea_hbm, eb_hbm, out_hbm,
                    aidx_v, bidx_v, rowsa_v, rowsb_v, ps_v, sema, semb):
    wid = _worker_id()

    def chunk(c, carry):
        off = wid * L_TILE + c * CHUNK
        pltpu.sync_copy(ea_hbm.at[pl.ds(off, CHUNK)], aidx_v)
        pltpu.sync_copy(eb_hbm.at[pl.ds(off, CHUNK)], bidx_v)
        cpa = pltpu.async_copy(z_hbm.at[aidx_v], rowsa_v, sema)
        cpb = pltpu.async_copy(z_hbm.at[bidx_v], rowsb_v, semb)
        cpa.wait()
        cpb.wait()

        def edge(e, carry2):
            acc = rowsa_v[e, pl.ds(0, 16)] * rowsb_v[e, pl.ds(0, 16)]
            for j in range(1, H // 16):
                acc = acc + rowsa_v[e, pl.ds(j * 16, 16)] * rowsb_v[e, pl.ds(j * 16, 16)]
            ps_v[e, :] = acc
            return carry2

        lax.fori_loop(0, CHUNK, edge, 0)
        pltpu.sync_copy(ps_v, out_hbm.at[pl.ds(off, CHUNK)])
        return carry

    lax.fori_loop(0, L_TILE // CHUNK, chunk, 0)


_sc_decode = pl.kernel(
    _sc_decode_body,
    out_type=jax.ShapeDtypeStruct((EL_PAD, 16), jnp.float32),
    mesh=plsc.VectorSubcoreMesh(core_axis_name="c", subcore_axis_name="s"),
    scratch_types=[
        pltpu.VMEM((CHUNK,), jnp.int32),
        pltpu.VMEM((CHUNK,), jnp.int32),
        pltpu.VMEM((CHUNK, H), jnp.float32),
        pltpu.VMEM((CHUNK, H), jnp.float32),
        pltpu.VMEM((CHUNK, 16), jnp.float32),
        pltpu.SemaphoreType.DMA,
        pltpu.SemaphoreType.DMA,
    ],
)


ROW_BLK = 512
GRID_N = N_PAD // ROW_BLK


def _dis_block(deg_blk):
    deg = 1.0 + deg_blk[0, :, 0:1] + deg_blk[1, :, 0:1]
    return lax.rsqrt(deg)


def _tc_prep1_body(x_ref, w_ref, deg_ref, g_ref):
    dis = _dis_block(deg_ref[...])
    h = jnp.dot(x_ref[...], w_ref[...], preferred_element_type=jnp.float32)
    g_ref[...] = h * dis


_tc_prep1 = pl.pallas_call(
    _tc_prep1_body,
    out_shape=jax.ShapeDtypeStruct((N_PAD, H), jnp.float32),
    grid=(GRID_N,),
    in_specs=[
        pl.BlockSpec((ROW_BLK, D), lambda i: (i, 0)),
        pl.BlockSpec((D, H), lambda i: (0, 0)),
        pl.BlockSpec((2, ROW_BLK, 8), lambda i: (0, i, 0)),
    ],
    out_specs=pl.BlockSpec((ROW_BLK, H), lambda i: (i, 0)),
)


def _tc_prep2_body(p_ref, g_ref, deg_ref, b_ref, w_ref, o_ref):
    dis = _dis_block(deg_ref[...])
    p = p_ref[...]
    z = jnp.maximum(dis * (p[0] + p[1] + g_ref[...]) + b_ref[...], 0.0)
    o_ref[...] = jnp.dot(z, w_ref[...], preferred_element_type=jnp.float32) * dis


_tc_prep2 = pl.pallas_call(
    _tc_prep2_body,
    out_shape=jax.ShapeDtypeStruct((N_PAD, H), jnp.float32),
    grid=(GRID_N,),
    in_specs=[
        pl.BlockSpec((2, ROW_BLK, H), lambda i: (0, i, 0)),
        pl.BlockSpec((ROW_BLK, H), lambda i: (i, 0)),
        pl.BlockSpec((2, ROW_BLK, 8), lambda i: (0, i, 0)),
        pl.BlockSpec((1, H), lambda i: (0, 0)),
        pl.BlockSpec((H, H), lambda i: (0, 0)),
    ],
    out_specs=pl.BlockSpec((ROW_BLK, H), lambda i: (i, 0)),
)


def _tc_final_body(p_ref, g_ref, deg_ref, b_ref, o_ref):
    dis = _dis_block(deg_ref[...])
    p = p_ref[...]
    o_ref[...] = dis * (p[0] + p[1] + g_ref[...]) + b_ref[...]


_tc_final = pl.pallas_call(
    _tc_final_body,
    out_shape=jax.ShapeDtypeStruct((N_PAD, H), jnp.float32),
    grid=(GRID_N,),
    in_specs=[
        pl.BlockSpec((2, ROW_BLK, H), lambda i: (0, i, 0)),
        pl.BlockSpec((ROW_BLK, H), lambda i: (i, 0)),
        pl.BlockSpec((2, ROW_BLK, 8), lambda i: (0, i, 0)),
        pl.BlockSpec((1, H), lambda i: (0, 0)),
    ],
    out_specs=pl.BlockSpec((ROW_BLK, H), lambda i: (i, 0)),
)


RED_BLK = 16
RED_ROWS = EL_PAD // 128


def _tc_reduce_body(ps_ref, o_ref):
    o_ref[...] = jnp.sum(ps_ref[...], axis=-1)


_tc_reduce = pl.pallas_call(
    _tc_reduce_body,
    out_shape=jax.ShapeDtypeStruct((RED_ROWS, 128), jnp.float32),
    grid=(RED_ROWS // RED_BLK,),
    in_specs=[pl.BlockSpec((RED_BLK, 128, 16), lambda i: (i, 0, 0))],
    out_specs=pl.BlockSpec((RED_BLK, 128), lambda i: (i, 0)),
)


def kernel(x, edge_index, edge_label_index, W1, b1, W2, b2):
    src, dst = edge_index[0], edge_index[1]
    epad = N + (jnp.arange(E_PAD - E, dtype=jnp.int32) % PAD_ROWS)
    dstp = jnp.concatenate([dst, epad])
    srcp = jnp.concatenate([src, epad])
    pk = (srcp | (dstp << 16)).reshape(NW, M_CH, MCHUNK)
    lpad = N + (jnp.arange(EL_PAD - EL, dtype=jnp.int32) % PAD_ROWS)
    ea = jnp.concatenate([edge_label_index[0], lpad])
    eb = jnp.concatenate([edge_label_index[1], lpad])
    x_pad = jnp.zeros((N_PAD, D), jnp.float32).at[:N].set(x)

    degparts = _sc_deg(dstp.reshape(NW * E_CH, CHUNK)).reshape(NC, N_PAD, H)[:, :, :8]
    g1 = _tc_prep1(x_pad, W1, degparts)
    p1 = _sc_msg(g1, pk).reshape(NC, N_PAD, H)
    g2 = _tc_prep2(p1, g1, degparts, b1.reshape(1, H), W2)
    p2 = _sc_msg(g2, pk).reshape(NC, N_PAD, H)
    z2 = _tc_final(p2, g2, degparts, b2.reshape(1, H))
    ps = _sc_decode(z2, ea, eb)
    red = _tc_reduce(ps.reshape(RED_ROWS, 128, 16))
    return red.reshape(EL_PAD)[:EL]


# speedup vs baseline: 21.2979x; 1.1201x over previous
import functools

import jax
import jax.numpy as jnp
from jax import lax
from jax.experimental import pallas as pl
from jax.experimental.pallas import tpu as pltpu
from jax.experimental.pallas import tpu_sc as plsc

N = 10000
D = 128
H = 128
E = 320000
EL = 100000

N_PAD = 10240
PAD_ROWS = N_PAD - N

NC, NS = 2, 16
NW = NC * NS
CHUNK = 128

E_TILE = 10240
E_PAD = NW * E_TILE
E_CH = E_TILE // CHUNK
L_CH = 26
L_TILE = L_CH * CHUNK
EL_PAD = NW * L_TILE

ROWS_PER_TILE = N_PAD // NS

MCHUNK = 128
M_CH = 80
M_TILE = M_CH * MCHUNK
M_PAD = NW * M_TILE


def _worker_id():
    return lax.axis_index("s") * NC + lax.axis_index("c")


def _fill_const(buf, val):
    v = jnp.full((16,), val, jnp.float32)
    n = buf.shape[0]

    def row(i, carry):
        for j in range(H // 16):
            buf[i, pl.ds(j * 16, 16)] = v
        return carry

    lax.fori_loop(0, n, row, 0)


def _zero_acc_slice(buf, acc_sh, sid):
    for j in range(ROWS_PER_TILE // CHUNK):
        pltpu.sync_copy(buf, acc_sh.at[pl.ds(sid * ROWS_PER_TILE + j * CHUNK, CHUNK)])


def _row_iota(buf, base, n):
    for j in range(n // 16):
        buf[pl.ds(j * 16, 16)] = base + j * 16 + lax.iota(jnp.int32, 16)


def _sc_deg_body(dst_hbm, out_hbm, ridx_v, didx_v, ones_v, deg_sh, sem):
    cid = lax.axis_index("c")
    sid = lax.axis_index("s")
    wid = _worker_id()
    _fill_const(ones_v, 0.0)
    _zero_acc_slice(ones_v, deg_sh, sid)
    _fill_const(ones_v, 1.0)
    _row_iota(ridx_v, wid * E_CH, E_CH)
    pltpu.async_copy(dst_hbm.at[ridx_v], didx_v, sem).wait()
    plsc.subcore_barrier()

    GRP = 4

    def group(g, carry):
        for k in range(GRP):
            pltpu.async_copy(ones_v, deg_sh.at[didx_v.at[g * GRP + k]], sem, add=True)
        for k in range(GRP):
            pltpu.make_async_copy(ones_v, deg_sh.at[didx_v.at[0]], sem).wait()
        return carry

    lax.fori_loop(0, E_CH // GRP, group, 0)
    plsc.subcore_barrier()
    pltpu.sync_copy(
        deg_sh.at[pl.ds(sid * ROWS_PER_TILE, ROWS_PER_TILE)],
        out_hbm.at[pl.ds(cid * N_PAD + sid * ROWS_PER_TILE, ROWS_PER_TILE)],
    )


_sc_deg = pl.kernel(
    _sc_deg_body,
    out_type=jax.ShapeDtypeStruct((NC * N_PAD, H), jnp.float32),
    mesh=plsc.VectorSubcoreMesh(core_axis_name="c", subcore_axis_name="s"),
    scratch_types=[
        pltpu.VMEM((E_CH,), jnp.int32),
        pltpu.VMEM((E_CH, CHUNK), jnp.int32),
        pltpu.VMEM((CHUNK, H), jnp.float32),
        pltpu.VMEM_SHARED((N_PAD, H), jnp.float32),
        pltpu.SemaphoreType.DMA,
    ],
)


def _sc_msg_body(g_hbm, pk_hbm, out_hbm,
                 pk_v, sidx0_v, sidx1_v, didx0_v, didx1_v, rows0_v, rows1_v,
                 acc_sh, semg0, semg1, sems0, sems1):
    cid = lax.axis_index("c")
    sid = lax.axis_index("s")
    wid = _worker_id()
    _fill_const(rows0_v, 0.0)
    _zero_acc_slice(rows0_v, acc_sh, sid)
    pltpu.sync_copy(pk_hbm.at[wid], pk_v)
    plsc.subcore_barrier()

    def unpack(c, sbuf, dbuf):
        for j in range(MCHUNK // 16):
            p = pk_v[c, pl.ds(j * 16, 16)]
            sbuf[pl.ds(j * 16, 16)] = p & 0xFFFF
            dbuf[pl.ds(j * 16, 16)] = lax.shift_right_logical(p, 16)

    def gather(sbuf, rbuf, sem):
        pltpu.async_copy(g_hbm.at[sbuf], rbuf, sem)

    def gwait(rbuf, sem):
        pltpu.make_async_copy(g_hbm.at[sidx0_v], rbuf, sem).wait()

    def sissue(dbuf, rbuf, sem):
        pltpu.async_copy(rbuf, acc_sh.at[dbuf], sem, add=True)

    def swait(sem):
        pltpu.make_async_copy(rows0_v, acc_sh.at[didx0_v], sem).wait()

    unpack(0, sidx0_v, didx0_v)
    gather(sidx0_v, rows0_v, semg0)

    def pair(i, carry):
        c0 = 2 * i

        @pl.when(i > 0)
        def _():
            swait(sems1)

        unpack(c0 + 1, sidx1_v, didx1_v)
        gather(sidx1_v, rows1_v, semg1)
        gwait(rows0_v, semg0)
        sissue(didx0_v, rows0_v, sems0)
        gwait(rows1_v, semg1)
        swait(sems0)

        @pl.when(i < M_CH // 2 - 1)
        def _():
            unpack(c0 + 2, sidx0_v, didx0_v)
            gather(sidx0_v, rows0_v, semg0)

        sissue(didx1_v, rows1_v, sems1)
        return carry

    lax.fori_loop(0, M_CH // 2, pair, 0)
    swait(sems1)
    plsc.subcore_barrier()
    pltpu.sync_copy(
        acc_sh.at[pl.ds(sid * ROWS_PER_TILE, ROWS_PER_TILE)],
        out_hbm.at[pl.ds(cid * N_PAD + sid * ROWS_PER_TILE, ROWS_PER_TILE)],
    )


_sc_msg = pl.kernel(
    _sc_msg_body,
    out_type=jax.ShapeDtypeStruct((NC * N_PAD, H), jnp.float32),
    mesh=plsc.VectorSubcoreMesh(core_axis_name="c", subcore_axis_name="s"),
    scratch_types=[
        pltpu.VMEM((M_CH, MCHUNK), jnp.int32),
        pltpu.VMEM((MCHUNK,), jnp.int32),
        pltpu.VMEM((MCHUNK,), jnp.int32),
        pltpu.VMEM((MCHUNK,), jnp.int32),
        pltpu.VMEM((MCHUNK,), jnp.int32),
        pltpu.VMEM((MCHUNK, H), jnp.float32),
        pltpu.VMEM((MCHUNK, H), jnp.float32),
        pltpu.VMEM_SHARED((N_PAD, H), jnp.float32),
        pltpu.SemaphoreType.DMA,
        pltpu.SemaphoreType.DMA,
        pltpu.SemaphoreType.DMA,
        pltpu.SemaphoreType.DMA,
    ],
)


def _sc_decode_body(z_hbm, pk_hbm, out_hbm,
                    pk_v, aidx0_v, bidx0_v, aidx1_v, bidx1_v,
                    ra0_v, rb0_v, ra1_v, rb1_v, ps_v,
                    sa0, sb0, sa1, sb1):
    wid = _worker_id()
    pltpu.sync_copy(pk_hbm.at[wid], pk_v)

    def unpack(c, abuf, bbuf):
        for j in range(CHUNK // 16):
            p = pk_v[c, pl.ds(j * 16, 16)]
            abuf[pl.ds(j * 16, 16)] = p & 0xFFFF
            bbuf[pl.ds(j * 16, 16)] = lax.shift_right_logical(p, 16)

    def gathers(abuf, bbuf, ra, rb, sema, semb):
        pltpu.async_copy(z_hbm.at[abuf], ra, sema)
        pltpu.async_copy(z_hbm.at[bbuf], rb, semb)

    def gwaits(ra, rb, sema, semb):
        pltpu.make_async_copy(z_hbm.at[aidx0_v], ra, sema).wait()
        pltpu.make_async_copy(z_hbm.at[aidx0_v], rb, semb).wait()

    def compute(c, ra, rb):
        def edge(e, carry):
            acc = ra[e, pl.ds(0, 16)] * rb[e, pl.ds(0, 16)]
            for j in range(1, H // 16):
                acc = acc + ra[e, pl.ds(j * 16, 16)] * rb[e, pl.ds(j * 16, 16)]
            ps_v[e, :] = acc
            return carry

        lax.fori_loop(0, CHUNK, edge, 0)
        pltpu.sync_copy(ps_v, out_hbm.at[pl.ds(wid * L_TILE + c * CHUNK, CHUNK)])

    unpack(0, aidx0_v, bidx0_v)
    gathers(aidx0_v, bidx0_v, ra0_v, rb0_v, sa0, sb0)

    def pair(i, carry):
        c0 = 2 * i
        unpack(c0 + 1, aidx1_v, bidx1_v)
        gathers(aidx1_v, bidx1_v, ra1_v, rb1_v, sa1, sb1)
        gwaits(ra0_v, rb0_v, sa0, sb0)
        compute(c0, ra0_v, rb0_v)

        @pl.when(i < L_CH // 2 - 1)
        def _():
            unpack(c0 + 2, aidx0_v, bidx0_v)
            gathers(aidx0_v, bidx0_v, ra0_v, rb0_v, sa0, sb0)

        gwaits(ra1_v, rb1_v, sa1, sb1)
        compute(c0 + 1, ra1_v, rb1_v)
        return carry

    lax.fori_loop(0, L_CH // 2, pair, 0)


_sc_decode = pl.kernel(
    _sc_decode_body,
    out_type=jax.ShapeDtypeStruct((EL_PAD, 16), jnp.float32),
    mesh=plsc.VectorSubcoreMesh(core_axis_name="c", subcore_axis_name="s"),
    scratch_types=[
        pltpu.VMEM((L_CH, CHUNK), jnp.int32),
        pltpu.VMEM((CHUNK,), jnp.int32),
        pltpu.VMEM((CHUNK,), jnp.int32),
        pltpu.VMEM((CHUNK,), jnp.int32),
        pltpu.VMEM((CHUNK,), jnp.int32),
        pltpu.VMEM((CHUNK, H), jnp.float32),
        pltpu.VMEM((CHUNK, H), jnp.float32),
        pltpu.VMEM((CHUNK, H), jnp.float32),
        pltpu.VMEM((CHUNK, H), jnp.float32),
        pltpu.VMEM((CHUNK, 16), jnp.float32),
        pltpu.SemaphoreType.DMA,
        pltpu.SemaphoreType.DMA,
        pltpu.SemaphoreType.DMA,
        pltpu.SemaphoreType.DMA,
    ],
)


ROW_BLK = 512
GRID_N = N_PAD // ROW_BLK


def _dis_block(deg_blk):
    deg = 1.0 + deg_blk[0, :, 0:1] + deg_blk[1, :, 0:1]
    return lax.rsqrt(deg)


def _tc_prep1_body(x_ref, w_ref, deg_ref, g_ref):
    dis = _dis_block(deg_ref[...])
    h = jnp.dot(x_ref[...], w_ref[...], preferred_element_type=jnp.float32)
    g_ref[...] = h * dis


_tc_prep1 = pl.pallas_call(
    _tc_prep1_body,
    out_shape=jax.ShapeDtypeStruct((N_PAD, H), jnp.float32),
    grid=(GRID_N,),
    in_specs=[
        pl.BlockSpec((ROW_BLK, D), lambda i: (i, 0)),
        pl.BlockSpec((D, H), lambda i: (0, 0)),
        pl.BlockSpec((2, ROW_BLK, 8), lambda i: (0, i, 0)),
    ],
    out_specs=pl.BlockSpec((ROW_BLK, H), lambda i: (i, 0)),
)


def _tc_prep2_body(p_ref, g_ref, deg_ref, b_ref, w_ref, o_ref):
    dis = _dis_block(deg_ref[...])
    p = p_ref[...]
    z = jnp.maximum(dis * (p[0] + p[1] + g_ref[...]) + b_ref[...], 0.0)
    o_ref[...] = jnp.dot(z, w_ref[...], preferred_element_type=jnp.float32) * dis


_tc_prep2 = pl.pallas_call(
    _tc_prep2_body,
    out_shape=jax.ShapeDtypeStruct((N_PAD, H), jnp.float32),
    grid=(GRID_N,),
    in_specs=[
        pl.BlockSpec((2, ROW_BLK, H), lambda i: (0, i, 0)),
        pl.BlockSpec((ROW_BLK, H), lambda i: (i, 0)),
        pl.BlockSpec((2, ROW_BLK, 8), lambda i: (0, i, 0)),
        pl.BlockSpec((1, H), lambda i: (0, 0)),
        pl.BlockSpec((H, H), lambda i: (0, 0)),
    ],
    out_specs=pl.BlockSpec((ROW_BLK, H), lambda i: (i, 0)),
)


def _tc_final_body(p_ref, g_ref, deg_ref, b_ref, o_ref):
    dis = _dis_block(deg_ref[...])
    p = p_ref[...]
    o_ref[...] = dis * (p[0] + p[1] + g_ref[...]) + b_ref[...]


_tc_final = pl.pallas_call(
    _tc_final_body,
    out_shape=jax.ShapeDtypeStruct((N_PAD, H), jnp.float32),
    grid=(GRID_N,),
    in_specs=[
        pl.BlockSpec((2, ROW_BLK, H), lambda i: (0, i, 0)),
        pl.BlockSpec((ROW_BLK, H), lambda i: (i, 0)),
        pl.BlockSpec((2, ROW_BLK, 8), lambda i: (0, i, 0)),
        pl.BlockSpec((1, H), lambda i: (0, 0)),
    ],
    out_specs=pl.BlockSpec((ROW_BLK, H), lambda i: (i, 0)),
)


RED_BLK = 16
RED_ROWS = EL_PAD // 128


def _tc_reduce_body(ps_ref, o_ref):
    o_ref[...] = jnp.sum(ps_ref[...], axis=-1)


_tc_reduce = pl.pallas_call(
    _tc_reduce_body,
    out_shape=jax.ShapeDtypeStruct((RED_ROWS, 128), jnp.float32),
    grid=(RED_ROWS // RED_BLK,),
    in_specs=[pl.BlockSpec((RED_BLK, 128, 16), lambda i: (i, 0, 0))],
    out_specs=pl.BlockSpec((RED_BLK, 128), lambda i: (i, 0)),
)


def kernel(x, edge_index, edge_label_index, W1, b1, W2, b2):
    src, dst = edge_index[0], edge_index[1]
    epad = N + (jnp.arange(E_PAD - E, dtype=jnp.int32) % PAD_ROWS)
    dstp = jnp.concatenate([dst, epad])
    srcp = jnp.concatenate([src, epad])
    pk = (srcp | (dstp << 16)).reshape(NW, M_CH, MCHUNK)
    lpad = N + (jnp.arange(EL_PAD - EL, dtype=jnp.int32) % PAD_ROWS)
    ea = jnp.concatenate([edge_label_index[0], lpad])
    eb = jnp.concatenate([edge_label_index[1], lpad])
    pkl = (ea | (eb << 16)).reshape(NW, L_CH, CHUNK)
    x_pad = jnp.zeros((N_PAD, D), jnp.float32).at[:N].set(x)

    degparts = _sc_deg(dstp.reshape(NW * E_CH, CHUNK)).reshape(NC, N_PAD, H)[:, :, :8]
    g1 = _tc_prep1(x_pad, W1, degparts)
    p1 = _sc_msg(g1, pk).reshape(NC, N_PAD, H)
    g2 = _tc_prep2(p1, g1, degparts, b1.reshape(1, H), W2)
    p2 = _sc_msg(g2, pk).reshape(NC, N_PAD, H)
    z2 = _tc_final(p2, g2, degparts, b2.reshape(1, H))
    ps = _sc_decode(z2, pkl)
    red = _tc_reduce(ps.reshape(RED_ROWS, 128, 16))
    return red.reshape(EL_PAD)[:EL]
